# baseline (speedup 1.0000x reference)
import functools

import jax
import jax.numpy as jnp
import numpy as np
from jax.experimental import pallas as pl

N_FFT = 1024
HOP = 256
WIN = 1024
EPS = 1e-11
NFREQ = N_FFT // 2 + 1
TAPS = WIN // HOP
BM = 256


def _idft_matrix() -> np.ndarray:
    k = np.arange(NFREQ, dtype=np.float64)
    n = np.arange(N_FFT, dtype=np.float64)
    coef = np.full(NFREQ, 2.0)
    coef[0] = 1.0
    coef[NFREQ - 1] = 1.0
    m = (coef[None, :] * np.cos(2.0 * np.pi * np.outer(n, k) / N_FFT)) / N_FFT
    return m.astype(np.float32)


def _istft_kernel(t_total, p0_ref, p1_ref, m_ref, w_ref, out_ref):
    k = pl.program_id(0)
    a = m_ref[...] * w_ref[...]
    x = jnp.concatenate([p0_ref[...], p1_ref[...]], axis=2)
    b = x.shape[0]
    m_idx = k * BM + jax.lax.broadcasted_iota(jnp.int32, (1, BM), 1)
    wsq = (w_ref[...] * w_ref[...]).reshape(TAPS, HOP)
    env = jnp.zeros((HOP, BM), dtype=jnp.float32)
    masks = []
    for j in range(TAPS):
        t = m_idx - j
        valid = jnp.logical_and(t >= 0, t < t_total).astype(jnp.float32)
        masks.append(valid)
        env = env + valid * wsq[j][:, None]
    inv_env = 1.0 / (env + EPS)
    a16 = a.astype(jnp.bfloat16)
    for bi in range(b):
        xb16 = x[bi].astype(jnp.bfloat16)
        acc = jnp.zeros((HOP, BM), dtype=jnp.float32)
        for j in range(TAPS):
            xs = xb16[:, BM - j:2 * BM - j]
            aj = a16[j * HOP:(j + 1) * HOP, :]
            acc = acc + masks[j] * jax.lax.dot_general(
                aj, xs, (((1,), (0,)), ((), ())),
                preferred_element_type=jnp.float32)
        out_ref[bi, :, :] = acc * inv_env


@jax.jit
def kernel(spec, window):
    b, nfreq, t = spec.shape
    n_blocks = t + TAPS - 1
    n_chunks = pl.cdiv(n_blocks, BM)
    mpad = n_chunks * BM
    t_blocks = t // BM

    m = jnp.asarray(_idft_matrix())
    w2d = window.reshape(WIN, 1)

    def idx_lo(k):
        return (0, 0, jnp.clip(k - 1, 0, t_blocks - 1))

    def idx_hi(k):
        return (0, 0, jnp.clip(k, 0, t_blocks - 1))

    out = pl.pallas_call(
        functools.partial(_istft_kernel, t),
        grid=(n_chunks,),
        in_specs=[
            pl.BlockSpec((b, nfreq, BM), idx_lo),
            pl.BlockSpec((b, nfreq, BM), idx_hi),
            pl.BlockSpec((WIN, NFREQ), lambda k: (0, 0)),
            pl.BlockSpec((WIN, 1), lambda k: (0, 0)),
        ],
        out_specs=pl.BlockSpec((b, HOP, BM), lambda k: (0, 0, k)),
        out_shape=jax.ShapeDtypeStruct((b, HOP, mpad), jnp.float32),
    )(spec, spec, m, w2d)

    pad = (WIN - HOP) // 2
    y = jnp.swapaxes(out, 1, 2).reshape(b, mpad * HOP)
    return jax.lax.dynamic_slice(y, (0, pad), (b, (t - 1) * HOP + WIN - 2 * pad))


# speedup vs baseline: 1.1201x; 1.1201x over previous
import functools

import jax
import jax.numpy as jnp
import numpy as np
from jax.experimental import pallas as pl
from jax.experimental.pallas import tpu as pltpu

N_FFT = 1024
HOP = 256
WIN = 1024
EPS = 1e-11
NFREQ = N_FFT // 2 + 1
PAD = (WIN - HOP) // 2
BQ = 256
HALO = 128


def _idft_matrix() -> np.ndarray:
    k = np.arange(NFREQ, dtype=np.float64)
    n = np.arange(N_FFT, dtype=np.float64)
    coef = np.full(NFREQ, 2.0)
    coef[0] = 1.0
    coef[NFREQ - 1] = 1.0
    m = (coef[None, :] * np.cos(2.0 * np.pi * np.outer(n, k) / N_FFT)) / N_FFT
    return m.astype(np.float32)


def _istft_kernel(t_total, main_ref, hi_ref, m_ref, w_ref, out_ref, c_ref):
    k = pl.program_id(0)

    @pl.when(k == 0)
    def _init():
        c_ref[...] = jnp.zeros_like(c_ref)

    a = m_ref[...] * w_ref[...]
    wsq1 = w_ref[...] * w_ref[...]

    q_idx = k * BQ + jax.lax.broadcasted_iota(jnp.int32, (1, BQ), 1)
    masks = {}
    for d in range(-2, 3):
        t = q_idx - d
        masks[d] = jnp.logical_and(t >= 0, t < t_total).astype(jnp.float32)

    env_top = (masks[-1] * wsq1[128 + 0:128 + 128] + masks[0] * wsq1[384:512]
               + masks[1] * wsq1[640:768] + masks[2] * wsq1[896:1024])
    env_bot = (masks[-2] * wsq1[0:128] + masks[-1] * wsq1[256:384]
               + masks[0] * wsq1[512:640] + masks[1] * wsq1[768:896])
    inv_env = 1.0 / (jnp.concatenate([env_top, env_bot], axis=0) + EPS)

    b = main_ref.shape[0]
    for bi in range(b):
        x = jnp.concatenate([c_ref[bi], main_ref[bi], hi_ref[bi]], axis=1)
        acc = jnp.zeros((HOP, BQ), dtype=jnp.float32)
        for d in (-1, 0, 1):
            xs = x[:, 128 - d:384 - d]
            ad = a[384 + 256 * d:640 + 256 * d, :]
            acc = acc + masks[d] * jax.lax.dot_general(
                ad, xs, (((1,), (0,)), ((), ())),
                preferred_element_type=jnp.float32)
        top = acc[0:128, :] + masks[2] * jax.lax.dot_general(
            a[896:1024, :], x[:, 126:382], (((1,), (0,)), ((), ())),
            preferred_element_type=jnp.float32)
        bot = acc[128:256, :] + masks[-2] * jax.lax.dot_general(
            a[0:128, :], x[:, 130:386], (((1,), (0,)), ((), ())),
            preferred_element_type=jnp.float32)
        y = jnp.concatenate([top, bot], axis=0) * inv_env
        out_ref[bi] = y.T

        c_ref[bi] = main_ref[bi, :, BQ - HALO:]


@jax.jit
def kernel(spec, window):
    b, nfreq, t = spec.shape
    n_chunks = t // BQ
    t_halo_blocks = t // HALO

    m = jnp.asarray(_idft_matrix())
    w2d = window.reshape(WIN, 1)

    out = pl.pallas_call(
        functools.partial(_istft_kernel, t),
        grid=(n_chunks,),
        in_specs=[
            pl.BlockSpec((b, nfreq, BQ), lambda k: (0, 0, k)),
            pl.BlockSpec((b, nfreq, HALO),
                         lambda k: (0, 0,
                                    jnp.clip(2 * k + 2, 0, t // HALO - 1))),
            pl.BlockSpec((WIN, NFREQ), lambda k: (0, 0)),
            pl.BlockSpec((WIN, 1), lambda k: (0, 0)),
        ],
        out_specs=pl.BlockSpec((b, BQ, HOP), lambda k: (0, k, 0)),
        out_shape=jax.ShapeDtypeStruct((b, t, HOP), jnp.float32),
        scratch_shapes=[
            pltpu.VMEM((b, nfreq, HALO), jnp.float32),
        ],
    )(spec, spec, m, w2d)

    return out.reshape(b, t * HOP)
